# baseline (speedup 1.0000x reference)
import functools

import jax
import jax.numpy as jnp
from jax import lax
from jax.experimental import pallas as pl
from jax.experimental.pallas import tpu as pltpu
from jax.experimental.pallas import tpu_sc as plsc

B = 16384
D = 64
V = 1000000
L = 16
NC, NS = 2, 16
NW = NC * NS
RPW = B // NW
CH = 128
CB = 32768

_mesh = plsc.VectorSubcoreMesh(core_axis_name="c", subcore_axis_name="s")


def _wsum_body(ut_ref, it_ref, wu_ref, wi_ref, vu_ref, vi_ref):
    vu_ref[0, :] = jnp.sum(ut_ref[...] * wu_ref[...], axis=0)
    vi_ref[0, :] = jnp.sum(it_ref[...] * wi_ref[...], axis=0)


_NCB = (V + CB - 1) // CB

_wsum = pl.pallas_call(
    _wsum_body,
    grid=(_NCB,),
    in_specs=[
        pl.BlockSpec((D, CB), lambda j: (0, j)),
        pl.BlockSpec((D, CB), lambda j: (0, j)),
        pl.BlockSpec((D, 1), lambda j: (0, 0)),
        pl.BlockSpec((D, 1), lambda j: (0, 0)),
    ],
    out_specs=[
        pl.BlockSpec((1, CB), lambda j: (0, j)),
        pl.BlockSpec((1, CB), lambda j: (0, j)),
    ],
    out_shape=[
        jax.ShapeDtypeStruct((1, V), jnp.float32),
        jax.ShapeDtypeStruct((1, V), jnp.float32),
    ],
)


@functools.partial(
    pl.kernel,
    mesh=_mesh,
    out_type=jax.ShapeDtypeStruct((B,), jnp.float32),
    scratch_types=[
        pltpu.VMEM((RPW,), jnp.int32),
        pltpu.VMEM((RPW,), jnp.int32),
        pltpu.VMEM((RPW,), jnp.float32),
        pltpu.VMEM((RPW,), jnp.float32),
        pltpu.VMEM((L,), jnp.float32),
        pltpu.VMEM((RPW,), jnp.float32),
        pltpu.SemaphoreType.DMA,
    ],
)
def _gather_sc(uids, iids, vu, vi, bvec, out,
               uidx_v, iidx_v, gu_v, gi_v, b_v, out_v, sem):
    wid = lax.axis_index("s") * NC + lax.axis_index("c")
    base = wid * RPW
    pltpu.sync_copy(uids.at[pl.ds(base, RPW)], uidx_v)
    pltpu.sync_copy(iids.at[pl.ds(base, RPW)], iidx_v)
    pltpu.sync_copy(bvec, b_v)
    handles = []
    for c in range(RPW // CH):
        handles.append(pltpu.async_copy(
            vu.at[uidx_v.at[pl.ds(c * CH, CH)]],
            gu_v.at[pl.ds(c * CH, CH)], sem))
        handles.append(pltpu.async_copy(
            vi.at[iidx_v.at[pl.ds(c * CH, CH)]],
            gi_v.at[pl.ds(c * CH, CH)], sem))
    for h in handles:
        h.wait()
    bias = b_v[...]
    for k in range(RPW // L):
        out_v[pl.ds(k * L, L)] = (gu_v[pl.ds(k * L, L)]
                                  + gi_v[pl.ds(k * L, L)] + bias)
    pltpu.sync_copy(out_v, out.at[pl.ds(base, RPW)])


def kernel(user_ids, item_ids, user_table, item_table, fc_w, fc_b):
    ut_t = user_table.T
    it_t = item_table.T
    wu = fc_w[:D]
    wi = fc_w[D:]
    vu, vi = _wsum(ut_t, it_t, wu, wi)
    bvec = jnp.broadcast_to(fc_b, (L,))
    out = _gather_sc(user_ids, item_ids, vu.reshape(V), vi.reshape(V), bvec)
    return out.reshape(B, 1)
